# Initial kernel scaffold; baseline (speedup 1.0000x reference)
import jax
import jax.numpy as jnp
from jax.experimental import pallas as pl


def kernel(uv, tex):
    raise NotImplementedError("write your pallas kernel here")


# speedup vs baseline: 3.0906x; 3.0906x over previous
import functools

import jax
import jax.numpy as jnp
from jax import lax
from jax.experimental import pallas as pl
from jax.experimental.pallas import tpu as pltpu
from jax.experimental.pallas import tpu_sc as plsc

H = 1024
W = 1024
D = 32
N = 2097152

NC = 2
NS = 16
L = 16
NW = NC * NS
PPW = N // NW
C = 512
NCHUNK = PPW // C
SLAB = 128
NSLAB = C // SLAB
GPS = SLAB // L


def _make_sc_kernel():
    mesh = plsc.VectorSubcoreMesh(core_axis_name="c", subcore_axis_name="s")

    @functools.partial(
        pl.kernel,
        mesh=mesh,
        out_type=jax.ShapeDtypeStruct((N, D), jnp.float32),
        compiler_params=pltpu.CompilerParams(use_tc_tiling_on_sc=False),
        scratch_types=[
            pltpu.VMEM((C,), jnp.float32),
            pltpu.VMEM((C,), jnp.float32),
            pltpu.VMEM((NSLAB, SLAB), jnp.int32),
            pltpu.VMEM((NSLAB, SLAB), jnp.int32),
            pltpu.VMEM((NSLAB, SLAB), jnp.int32),
            pltpu.VMEM((NSLAB, SLAB), jnp.int32),
            pltpu.VMEM((C,), jnp.float32),
            pltpu.VMEM((C,), jnp.float32),
            pltpu.VMEM((C, D), jnp.float32),
            pltpu.VMEM((C, D), jnp.float32),
            pltpu.VMEM((C, D), jnp.float32),
            pltpu.VMEM((C, D), jnp.float32),
            pltpu.VMEM((C, D), jnp.float32),
            pltpu.SemaphoreType.DMA,
        ],
    )
    def theta_sc(ux_hbm, uy_hbm, tex_hbm, out_hbm,
                 ux_v, uy_v, idx00_v, idx01_v, idx10_v, idx11_v,
                 wx_v, wy_v, v00_v, v01_v, v10_v, v11_v, out_v, sem):
        cid = lax.axis_index("c")
        sid = lax.axis_index("s")
        wid = sid * NC + cid
        wbase = wid * PPW

        def chunk_body(g, carry):
            base = wbase + g * C
            pltpu.sync_copy(ux_hbm.at[pl.ds(base, C)], ux_v)
            pltpu.sync_copy(uy_hbm.at[pl.ds(base, C)], uy_v)

            def pha_slab(j, carry):
                def pha(ii, carry):
                    i = j * GPS + ii
                    u = ux_v[pl.ds(i * L, L)]
                    v = uy_v[pl.ds(i * L, L)]
                    ix = jnp.minimum(jnp.maximum(u * (W - 1.0), 0.0), W - 1.0)
                    iy = jnp.minimum(jnp.maximum(v * (H - 1.0), 0.0), H - 1.0)
                    x0 = jnp.minimum(ix.astype(jnp.int32), W - 2)
                    y0 = jnp.minimum(iy.astype(jnp.int32), H - 2)
                    wx = ix - x0.astype(jnp.float32)
                    wy = iy - y0.astype(jnp.float32)
                    i00 = y0 * W + x0
                    idx00_v[j, pl.ds(ii * L, L)] = i00
                    idx01_v[j, pl.ds(ii * L, L)] = i00 + 1
                    idx10_v[j, pl.ds(ii * L, L)] = i00 + W
                    idx11_v[j, pl.ds(ii * L, L)] = i00 + (W + 1)
                    wx_v[pl.ds(i * L, L)] = wx
                    wy_v[pl.ds(i * L, L)] = wy
                    return carry
                return lax.fori_loop(0, GPS, pha, carry)
            lax.fori_loop(0, NSLAB, pha_slab, 0)

            cps = []
            for j in range(NSLAB):
                dst = pl.ds(j * SLAB, SLAB)
                cps.append(pltpu.async_copy(
                    tex_hbm.at[idx00_v.at[j]], v00_v.at[dst], sem))
                cps.append(pltpu.async_copy(
                    tex_hbm.at[idx01_v.at[j]], v01_v.at[dst], sem))
                cps.append(pltpu.async_copy(
                    tex_hbm.at[idx10_v.at[j]], v10_v.at[dst], sem))
                cps.append(pltpu.async_copy(
                    tex_hbm.at[idx11_v.at[j]], v11_v.at[dst], sem))
            for cp in cps:
                cp.wait()

            def phc(i, carry):
                wxv = wx_v[pl.ds(i * L, L)]
                wyv = wy_v[pl.ds(i * L, L)]
                for p in range(L):
                    wxp = wxv[p]
                    wyp = wyv[p]
                    pt = i * L + p
                    for h in range(D // L):
                        s = pl.ds(h * L, L)
                        a = v00_v[pt, s]
                        b = v01_v[pt, s]
                        c = v10_v[pt, s]
                        d = v11_v[pt, s]
                        top = a + wxp * (b - a)
                        bot = c + wxp * (d - c)
                        out_v[pt, s] = top + wyp * (bot - top)
                return carry
            lax.fori_loop(0, C // L, phc, 0)

            pltpu.sync_copy(out_v, out_hbm.at[pl.ds(base, C)])
            return carry
        lax.fori_loop(0, NCHUNK, chunk_body, 0)

    return theta_sc


_THETA_SC = _make_sc_kernel()


def kernel(uv, tex):
    tex_t = jnp.transpose(tex[0], (1, 2, 0)).reshape(H * W, D)
    ux = uv[:, 0]
    uy = uv[:, 1]
    return _THETA_SC(ux, uy, tex_t)


# speedup vs baseline: 3.4140x; 1.1046x over previous
import functools

import jax
import jax.numpy as jnp
from jax import lax
from jax.experimental import pallas as pl
from jax.experimental.pallas import tpu as pltpu
from jax.experimental.pallas import tpu_sc as plsc

H = 1024
W = 1024
D = 32
N = 2097152

NC = 2
NS = 16
L = 16
NW = NC * NS
PPW = N // NW
C = 256
NCHUNK = PPW // C
SLAB = 128
NSLAB = C // SLAB
GPS = SLAB // L


def _make_sc_kernel():
    mesh = plsc.VectorSubcoreMesh(core_axis_name="c", subcore_axis_name="s")

    vbuf = lambda: pltpu.VMEM((C, D), jnp.float32)
    ibuf = lambda: pltpu.VMEM((NSLAB, SLAB), jnp.int32)
    fbuf = lambda: pltpu.VMEM((C,), jnp.float32)

    @functools.partial(
        pl.kernel,
        mesh=mesh,
        out_type=jax.ShapeDtypeStruct((N * D,), jnp.float32),
        compiler_params=pltpu.CompilerParams(use_tc_tiling_on_sc=False),
        scratch_types=[
            fbuf(), fbuf(),
            fbuf(), fbuf(),
            ibuf(), ibuf(), ibuf(), ibuf(),
            ibuf(), ibuf(), ibuf(), ibuf(),
            fbuf(), fbuf(),
            fbuf(), fbuf(),
            vbuf(), vbuf(), vbuf(), vbuf(),
            vbuf(), vbuf(), vbuf(), vbuf(),
            pltpu.VMEM((C * D,), jnp.float32),
            pltpu.SemaphoreType.DMA,
            pltpu.SemaphoreType.DMA,
        ],
    )
    def theta_sc(ux_hbm, uy_hbm, tex_hbm, out_hbm,
                 uxa_v, uya_v, uxb_v, uyb_v,
                 i00a, i01a, i10a, i11a, i00b, i01b, i10b, i11b,
                 wxa, wya, wxb, wyb,
                 v00a, v01a, v10a, v11a, v00b, v01b, v10b, v11b,
                 out_v, sem0, sem1):
        cid = lax.axis_index("c")
        sid = lax.axis_index("s")
        wid = sid * NC + cid
        wbase = wid * PPW

        bufs = (
            ((uxa_v, uya_v), (i00a, i01a, i10a, i11a), wxa, wya,
             (v00a, v01a, v10a, v11a), sem0),
            ((uxb_v, uyb_v), (i00b, i01b, i10b, i11b), wxb, wyb,
             (v00b, v01b, v10b, v11b), sem1),
        )

        def load_and_index(g, bset):
            (ux_v, uy_v), idxs, wx_v, wy_v, _, _ = bufs[bset]
            base = wbase + g * C
            pltpu.sync_copy(ux_hbm.at[pl.ds(base, C)], ux_v)
            pltpu.sync_copy(uy_hbm.at[pl.ds(base, C)], uy_v)

            def pha_slab(j, carry):
                def pha(ii, carry):
                    i = j * GPS + ii
                    u = ux_v[pl.ds(i * L, L)]
                    v = uy_v[pl.ds(i * L, L)]
                    ix = jnp.minimum(jnp.maximum(u * (W - 1.0), 0.0), W - 1.0)
                    iy = jnp.minimum(jnp.maximum(v * (H - 1.0), 0.0), H - 1.0)
                    x0 = jnp.minimum(ix.astype(jnp.int32), W - 2)
                    y0 = jnp.minimum(iy.astype(jnp.int32), H - 2)
                    wx = ix - x0.astype(jnp.float32)
                    wy = iy - y0.astype(jnp.float32)
                    i00 = y0 * W + x0
                    s = pl.ds(ii * L, L)
                    idxs[0][j, s] = i00
                    idxs[1][j, s] = i00 + 1
                    idxs[2][j, s] = i00 + W
                    idxs[3][j, s] = i00 + (W + 1)
                    wx_v[pl.ds(i * L, L)] = wx
                    wy_v[pl.ds(i * L, L)] = wy
                    return carry
                return lax.fori_loop(0, GPS, pha, carry)
            lax.fori_loop(0, NSLAB, pha_slab, 0)

        def fire(bset):
            _, idxs, _, _, vs, sem = bufs[bset]
            for j in range(NSLAB):
                dst = pl.ds(j * SLAB, SLAB)
                for q in range(4):
                    pltpu.async_copy(
                        tex_hbm.at[idxs[q].at[j]], vs[q].at[dst], sem)

        def drain(bset):
            _, idxs, _, _, vs, sem = bufs[bset]
            for j in range(NSLAB):
                dst = pl.ds(j * SLAB, SLAB)
                for q in range(4):
                    pltpu.make_async_copy(
                        tex_hbm.at[idxs[q].at[j]], vs[q].at[dst], sem).wait()

        def combine_store(g, bset):
            _, _, wx_v, wy_v, (v00, v01, v10, v11), _ = bufs[bset]
            base = wbase + g * C

            def phc(i, carry):
                wxv = wx_v[pl.ds(i * L, L)]
                wyv = wy_v[pl.ds(i * L, L)]
                for p in range(L):
                    wxp = wxv[p]
                    wyp = wyv[p]
                    pt = i * L + p
                    for h in range(D // L):
                        s = pl.ds(h * L, L)
                        a = v00[pt, s]
                        b = v01[pt, s]
                        c = v10[pt, s]
                        d = v11[pt, s]
                        top = a + wxp * (b - a)
                        bot = c + wxp * (d - c)
                        out_v[pl.ds(pt * D + h * L, L)] = (
                            top + wyp * (bot - top))
                return carry
            lax.fori_loop(0, C // L, phc, 0)
            pltpu.sync_copy(out_v, out_hbm.at[pl.ds(base * D, C * D)])

        load_and_index(0, 0)
        fire(0)

        def outer(gg, carry):
            for b in range(2):
                g = 2 * gg + b
                nxt = g + 1
                if b == 0:
                    load_and_index(nxt, 1)
                    fire(1)
                else:
                    @pl.when(gg < NCHUNK // 2 - 1)
                    def _():
                        load_and_index(nxt, 0)
                        fire(0)
                drain(b)
                combine_store(g, b)
            return carry
        lax.fori_loop(0, NCHUNK // 2, outer, 0)

    return theta_sc


_THETA_SC = _make_sc_kernel()


def kernel(uv, tex):
    tex_t = jnp.transpose(tex[0], (1, 2, 0)).reshape(H * W, D)
    ux = uv[:, 0]
    uy = uv[:, 1]
    out = _THETA_SC(ux, uy, tex_t)
    return out.reshape(N, D)
